# baseline (speedup 1.0000x reference)
import functools

import jax
import jax.numpy as jnp
from jax import lax
from jax.experimental import pallas as pl
from jax.experimental.pallas import tpu as pltpu
from jax.experimental.pallas import tpu_sc as plsc

BATCH = 16384
NUM_FIELDS = 26
VOCAB = 100000
EMBED_DIM = 16

NC = 2
NS = 16
LANES = 16
NW = NC * NS

R = NUM_FIELDS * EMBED_DIM
ROWS_PER_W = R // NW
BCHUNK = 4096
NBCHUNK = BATCH // BCHUNK
NCH = ROWS_PER_W * NBCHUNK
SLICES = BCHUNK // LANES
UNROLL = 16

SEG = 4
SEGLEN = 24960
_SEG_BOUNDS = [
    (s * SEGLEN, SEGLEN if s < SEG - 1 else VOCAB - (SEG - 1) * SEGLEN)
    for s in range(SEG)
]


def _body(tab_hbm, idx_hbm, out_hbm, row_v, idxf, out2, rsem, wsem):
    wid = lax.axis_index("s") * NC + lax.axis_index("c")
    r0 = wid * ROWS_PER_W
    zeros16 = lax.broadcasted_iota(jnp.int32, (LANES,), 0) * 0

    def row_copy(k):
        return [
            pltpu.async_copy(
                tab_hbm.at[pl.ds(r0 + k, 1), pl.ds(v0, vl)],
                row_v.at[:, pl.ds(v0, vl)],
                rsem,
            )
            for v0, vl in _SEG_BOUNDS
        ]

    def row_wait(hs):
        for h in hs:
            h.wait()

    def idx_load(f):
        pltpu.sync_copy(idx_hbm.at[pl.ds(f * BATCH, BATCH)], idxf)

    idx_load(r0 // EMBED_DIM)
    rh = row_copy(0)
    row_wait(rh)
    wh = [None] * NCH
    for k in range(ROWS_PER_W):
        f = (r0 + k) // EMBED_DIM
        if k > 0:
            f_prev = (r0 + k - 1) // EMBED_DIM

            @pl.when(f != f_prev)
            def _():
                idx_load(f)

        for c in range(NBCHUNK):
            t = k * NBCHUNK + c
            cur = t & 1
            if t >= 2:
                wh[t - 2].wait()

            @plsc.parallel_loop(0, SLICES, step=1, unroll=UNROLL)
            def _gather(j, c=c, cur=cur):
                off = j * LANES
                vidx = idxf[pl.ds(c * BCHUNK + off, LANES)]
                out2[cur, pl.ds(off, LANES)] = plsc.load_gather(
                    row_v, [zeros16, vidx]
                )

            if c == NBCHUNK - 1 and k + 1 < ROWS_PER_W:
                rh = row_copy(k + 1)
            wh[t] = pltpu.async_copy(
                out2.at[pl.ds(cur, 1), :],
                out_hbm.at[pl.ds(r0 + k, 1), pl.ds(c * BCHUNK, BCHUNK)],
                wsem,
            )
            if c == NBCHUNK - 1 and k + 1 < ROWS_PER_W:
                row_wait(rh)
    wh[NCH - 2].wait()
    wh[NCH - 1].wait()


@functools.lru_cache(maxsize=1)
def _gather_kernel():
    return functools.partial(
        pl.kernel,
        out_type=jax.ShapeDtypeStruct((R, BATCH), jnp.float32),
        mesh=plsc.VectorSubcoreMesh(
            core_axis_name="c", subcore_axis_name="s", num_cores=NC, num_subcores=NS
        ),
        scratch_types=[
            pltpu.VMEM((1, VOCAB), jnp.float32),
            pltpu.VMEM((BATCH,), jnp.int32),
            pltpu.VMEM((2, BCHUNK), jnp.float32),
            pltpu.SemaphoreType.DMA,
            pltpu.SemaphoreType.DMA,
        ],
        compiler_params=pltpu.CompilerParams(
            use_tc_tiling_on_sc=True, needs_layout_passes=False
        ),
    )(_body)


def kernel(indices, tables):
    tab2d = jnp.transpose(tables, (0, 2, 1)).reshape(R, VOCAB)
    idx_lin = jnp.transpose(indices.astype(jnp.int32), (1, 0)).reshape(
        NUM_FIELDS * BATCH
    )
    out_t = _gather_kernel()(tab2d, idx_lin)
    return jnp.transpose(out_t, (1, 0))


# speedup vs baseline: 1.0220x; 1.0220x over previous
import functools

import jax
import jax.numpy as jnp
from jax import lax
from jax.experimental import pallas as pl
from jax.experimental.pallas import tpu as pltpu
from jax.experimental.pallas import tpu_sc as plsc

BATCH = 16384
NUM_FIELDS = 26
VOCAB = 100000
EMBED_DIM = 16

NC = 2
NS = 16
LANES = 16
NW = NC * NS

R = NUM_FIELDS * EMBED_DIM
ROWS_PER_W = R // NW
BCHUNK = 4096
NBCHUNK = BATCH // BCHUNK
NCH = ROWS_PER_W * NBCHUNK
SLICES = BCHUNK // LANES
UNROLL = 8

SEG = 4
SEGLEN = 24960
_SEG_BOUNDS = [
    (s * SEGLEN, SEGLEN if s < SEG - 1 else VOCAB - (SEG - 1) * SEGLEN)
    for s in range(SEG)
]


def _body(tab_hbm, idx_hbm, out_hbm, row_v, idxf, out2, rsem, wsem):
    wid = lax.axis_index("s") * NC + lax.axis_index("c")
    r0 = wid * ROWS_PER_W
    zeros16 = lax.broadcasted_iota(jnp.int32, (LANES,), 0) * 0

    def row_copy(k):
        return [
            pltpu.async_copy(
                tab_hbm.at[pl.ds(r0 + k, 1), pl.ds(v0, vl)],
                row_v.at[:, pl.ds(v0, vl)],
                rsem,
            )
            for v0, vl in _SEG_BOUNDS
        ]

    def row_wait(hs):
        for h in hs:
            h.wait()

    def idx_load(f):
        pltpu.sync_copy(idx_hbm.at[pl.ds(f * BATCH, BATCH)], idxf)

    idx_load(r0 // EMBED_DIM)
    rh = row_copy(0)
    row_wait(rh)
    wh = [None] * NCH
    for k in range(ROWS_PER_W):
        f = (r0 + k) // EMBED_DIM
        if k > 0:
            f_prev = (r0 + k - 1) // EMBED_DIM

            @pl.when(f != f_prev)
            def _():
                idx_load(f)

        for c in range(NBCHUNK):
            t = k * NBCHUNK + c
            cur = t & 1
            if t >= 2:
                wh[t - 2].wait()

            @plsc.parallel_loop(0, SLICES, step=1, unroll=UNROLL)
            def _gather(j, c=c, cur=cur):
                off = j * LANES
                vidx = idxf[pl.ds(c * BCHUNK + off, LANES)]
                out2[cur, pl.ds(off, LANES)] = plsc.load_gather(
                    row_v, [zeros16, vidx]
                )

            if c == NBCHUNK - 1 and k + 1 < ROWS_PER_W:
                rh = row_copy(k + 1)
            wh[t] = pltpu.async_copy(
                out2.at[pl.ds(cur, 1), :],
                out_hbm.at[pl.ds(r0 + k, 1), pl.ds(c * BCHUNK, BCHUNK)],
                wsem,
            )
            if c == NBCHUNK - 1 and k + 1 < ROWS_PER_W:
                row_wait(rh)
    wh[NCH - 2].wait()
    wh[NCH - 1].wait()


@functools.lru_cache(maxsize=1)
def _gather_kernel():
    return functools.partial(
        pl.kernel,
        out_type=jax.ShapeDtypeStruct((R, BATCH), jnp.float32),
        mesh=plsc.VectorSubcoreMesh(
            core_axis_name="c", subcore_axis_name="s", num_cores=NC, num_subcores=NS
        ),
        scratch_types=[
            pltpu.VMEM((1, VOCAB), jnp.float32),
            pltpu.VMEM((BATCH,), jnp.int32),
            pltpu.VMEM((2, BCHUNK), jnp.float32),
            pltpu.SemaphoreType.DMA,
            pltpu.SemaphoreType.DMA,
        ],
        compiler_params=pltpu.CompilerParams(
            use_tc_tiling_on_sc=True, needs_layout_passes=False
        ),
    )(_body)


def kernel(indices, tables):
    tab2d = jnp.transpose(tables, (0, 2, 1)).reshape(R, VOCAB)
    idx_lin = jnp.transpose(indices.astype(jnp.int32), (1, 0)).reshape(
        NUM_FIELDS * BATCH
    )
    out_t = _gather_kernel()(tab2d, idx_lin)
    return jnp.transpose(out_t, (1, 0))
